# baseline (speedup 1.0000x reference)
import functools

import jax
import jax.numpy as jnp
from jax import lax
from jax.experimental import pallas as pl
from jax.experimental.pallas import tpu as pltpu
from jax.experimental.pallas import tpu_sc as plsc

TOPK = 8
NG = 64
D = 2048
ROWS = 16384
NW = 32
BR = 1024
NCHUNK = 1
CH = ROWS // NCHUNK


def _scores_block(x_ref, w_ref, b_ref, s_ref):
    logits = jnp.dot(x_ref[...], w_ref[...], preferred_element_type=jnp.float32)
    logits = logits + b_ref[...]
    e = jnp.exp(logits - jnp.max(logits, axis=1, keepdims=True))
    s_ref[:, 0:NG] = e / jnp.sum(e, axis=1, keepdims=True)


def _scores_tc_chunk(hs, wt, b2, c):
    off = c * (CH // BR)
    return pl.pallas_call(
        _scores_block,
        grid=(CH // BR,),
        in_specs=[
            pl.BlockSpec((BR, D), lambda ii: (ii + off, 0)),
            pl.BlockSpec((D, NG), lambda ii: (0, 0)),
            pl.BlockSpec((1, NG), lambda ii: (0, 0)),
        ],
        out_specs=pl.BlockSpec((BR, 2 * NG), lambda ii: (ii, 0)),
        out_shape=jax.ShapeDtypeStruct((CH, 2 * NG), jnp.float32),
    )(hs, wt, b2)


def _merge(ka, va, kb, vb):
    kbr = lax.rev(kb, (0,))
    vbr = lax.rev(vb, (0,))
    take_a = ka >= kbr
    kk = jnp.where(take_a, ka, kbr)
    vv = jnp.where(take_a, va, vbr)
    return plsc.sort_key_val(kk, vv, descending=True)


def _make_sc_topk(rows):
    rpw = rows // NW

    def body(s_hbm, idx_hbm, w_hbm, sv, iv, wv):
        wid = lax.axis_index("s") * 2 + lax.axis_index("c")
        pltpu.sync_copy(s_hbm.at[pl.ds(wid * rpw, rpw)], sv)
        lane = lax.iota(jnp.int32, 16)
        lane_f = lane.astype(jnp.float32)
        lt8 = lane < 8

        @plsc.parallel_loop(0, rpw, step=1, unroll=4)
        def row(r):
            ks, vs = [], []
            for j in range(4):
                sc = sv[r, pl.ds(16 * j, 16)]
                ij = lane + (16 * j)
                kj = jnp.where(sc == 0.0, (-1.0 - 16.0 * j) - lane_f, sc)
                kj, ij = plsc.sort_key_val(kj, ij, descending=True)
                ks.append(kj)
                vs.append(ij)
            k01, i01 = _merge(ks[0], vs[0], ks[1], vs[1])
            k23, i23 = _merge(ks[2], vs[2], ks[3], vs[3])
            kt, it = _merge(k01, i01, k23, i23)
            wt = jnp.maximum(kt, 0.0)
            denom = jnp.sum(jnp.where(lt8, wt, 0.0)) + 1e-20
            plsc.store_compressed(iv.at[pl.ds(r * 8, 16)], it, mask=lt8)
            plsc.store_compressed(wv.at[pl.ds(r * 8, 16)], wt / denom, mask=lt8)

        pltpu.sync_copy(iv.at[pl.ds(0, rpw * 8)],
                        idx_hbm.at[pl.ds(wid * (rpw * 8), rpw * 8)])
        pltpu.sync_copy(wv.at[pl.ds(0, rpw * 8)],
                        w_hbm.at[pl.ds(wid * (rpw * 8), rpw * 8)])

    return pl.kernel(
        body,
        mesh=plsc.VectorSubcoreMesh(core_axis_name="c", subcore_axis_name="s"),
        out_type=[
            jax.ShapeDtypeStruct((rows * 8,), jnp.int32),
            jax.ShapeDtypeStruct((rows * 8,), jnp.float32),
        ],
        scratch_types=[
            pltpu.VMEM((rpw, 128), jnp.float32),
            pltpu.VMEM((rpw * 8 + 8,), jnp.int32),
            pltpu.VMEM((rpw * 8 + 8,), jnp.float32),
        ],
        compiler_params=pltpu.CompilerParams(needs_layout_passes=False),
    )


_sc_topk_chunk = _make_sc_topk(CH)


def kernel(x, weight, bias):
    bsz, seq_len, h = x.shape
    hs = x.reshape(-1, h)
    wt = weight.T
    b2 = bias.reshape(1, NG)
    idx_parts, w_parts = [], []
    for c in range(NCHUNK):
        scores = _scores_tc_chunk(hs, wt, b2, c)
        idx_flat, w_flat = _sc_topk_chunk(scores)
        idx_parts.append(idx_flat)
        w_parts.append(w_flat)
    if NCHUNK == 1:
        idx = idx_parts[0].reshape(ROWS, TOPK)
        w = w_parts[0].reshape(ROWS, TOPK)
    else:
        idx = jnp.concatenate(idx_parts).reshape(ROWS, TOPK)
        w = jnp.concatenate(w_parts).reshape(ROWS, TOPK)
    aux_loss = jnp.zeros((), dtype=jnp.float32)
    return (idx, w, aux_loss)


# speedup vs baseline: 1.1092x; 1.1092x over previous
import functools

import jax
import jax.numpy as jnp
from jax import lax
from jax.experimental import pallas as pl
from jax.experimental.pallas import tpu as pltpu
from jax.experimental.pallas import tpu_sc as plsc

TOPK = 8
NG = 64
D = 2048
ROWS = 16384
NW = 32
BR = 1024
NCHUNK = 2
CH = ROWS // NCHUNK


def _scores_block(x_ref, w_ref, b_ref, s_ref):
    logits = jnp.dot(x_ref[...], w_ref[...], preferred_element_type=jnp.float32)
    logits = logits + b_ref[...]
    e = jnp.exp(logits - jnp.max(logits, axis=1, keepdims=True))
    s_ref[:, 0:NG] = e / jnp.sum(e, axis=1, keepdims=True)


def _scores_tc_chunk(hs, wt, b2, c):
    off = c * (CH // BR)
    return pl.pallas_call(
        _scores_block,
        grid=(CH // BR,),
        in_specs=[
            pl.BlockSpec((BR, D), lambda ii: (ii + off, 0)),
            pl.BlockSpec((D, NG), lambda ii: (0, 0)),
            pl.BlockSpec((1, NG), lambda ii: (0, 0)),
        ],
        out_specs=pl.BlockSpec((BR, 2 * NG), lambda ii: (ii, 0)),
        out_shape=jax.ShapeDtypeStruct((CH, 2 * NG), jnp.float32),
    )(hs, wt, b2)


def _merge(ka, va, kb, vb):
    kbr = lax.rev(kb, (0,))
    vbr = lax.rev(vb, (0,))
    take_a = ka >= kbr
    kk = jnp.where(take_a, ka, kbr)
    vv = jnp.where(take_a, va, vbr)
    return plsc.sort_key_val(kk, vv, descending=True)


def _make_sc_topk(rows):
    rpw = rows // NW

    def body(s_hbm, idx_hbm, w_hbm, sv, iv, wv):
        wid = lax.axis_index("s") * 2 + lax.axis_index("c")
        pltpu.sync_copy(s_hbm.at[pl.ds(wid * rpw, rpw)], sv)
        lane = lax.iota(jnp.int32, 16)
        lane_f = lane.astype(jnp.float32)
        lt8 = lane < 8

        @plsc.parallel_loop(0, rpw, step=1, unroll=4)
        def row(r):
            ks, vs = [], []
            for j in range(4):
                sc = sv[r, pl.ds(16 * j, 16)]
                ij = lane + (16 * j)
                kj = jnp.where(sc == 0.0, (-1.0 - 16.0 * j) - lane_f, sc)
                kj, ij = plsc.sort_key_val(kj, ij, descending=True)
                ks.append(kj)
                vs.append(ij)
            k01, i01 = _merge(ks[0], vs[0], ks[1], vs[1])
            k23, i23 = _merge(ks[2], vs[2], ks[3], vs[3])
            kt, it = _merge(k01, i01, k23, i23)
            wt = jnp.maximum(kt, 0.0)
            denom = jnp.sum(jnp.where(lt8, wt, 0.0)) + 1e-20
            plsc.store_compressed(iv.at[pl.ds(r * 8, 16)], it, mask=lt8)
            plsc.store_compressed(wv.at[pl.ds(r * 8, 16)], wt / denom, mask=lt8)

        pltpu.sync_copy(iv.at[pl.ds(0, rpw * 8)],
                        idx_hbm.at[pl.ds(wid * (rpw * 8), rpw * 8)])
        pltpu.sync_copy(wv.at[pl.ds(0, rpw * 8)],
                        w_hbm.at[pl.ds(wid * (rpw * 8), rpw * 8)])

    return pl.kernel(
        body,
        mesh=plsc.VectorSubcoreMesh(core_axis_name="c", subcore_axis_name="s"),
        out_type=[
            jax.ShapeDtypeStruct((rows * 8,), jnp.int32),
            jax.ShapeDtypeStruct((rows * 8,), jnp.float32),
        ],
        scratch_types=[
            pltpu.VMEM((rpw, 128), jnp.float32),
            pltpu.VMEM((rpw * 8 + 8,), jnp.int32),
            pltpu.VMEM((rpw * 8 + 8,), jnp.float32),
        ],
        compiler_params=pltpu.CompilerParams(needs_layout_passes=False),
    )


_sc_topk_chunk = _make_sc_topk(CH)


def kernel(x, weight, bias):
    bsz, seq_len, h = x.shape
    hs = x.reshape(-1, h)
    wt = weight.T
    b2 = bias.reshape(1, NG)
    idx_parts, w_parts = [], []
    for c in range(NCHUNK):
        scores = _scores_tc_chunk(hs, wt, b2, c)
        idx_flat, w_flat = _sc_topk_chunk(scores)
        idx_parts.append(idx_flat)
        w_parts.append(w_flat)
    if NCHUNK == 1:
        idx = idx_parts[0].reshape(ROWS, TOPK)
        w = w_parts[0].reshape(ROWS, TOPK)
    else:
        idx = jnp.concatenate(idx_parts).reshape(ROWS, TOPK)
        w = jnp.concatenate(w_parts).reshape(ROWS, TOPK)
    aux_loss = jnp.zeros((), dtype=jnp.float32)
    return (idx, w, aux_loss)
